# Initial kernel scaffold; baseline (speedup 1.0000x reference)
import jax
import jax.numpy as jnp
from jax.experimental import pallas as pl


def kernel(x, pos, edge_index, batch, l1w0, l1b0, l1w1, l1b1, l1w2, l1b2, g1w0, g1b0, g1w1, g1b1, g1w2, g1b2, l2w0, l2b0, l2w1, l2b1, l2w2, l2b2, g2w0, g2b0, g2w1, g2b1, g2w2, g2b2, linw, linb):
    raise NotImplementedError("write your pallas kernel here")


# speedup vs baseline: 1.0028x; 1.0028x over previous
import jax
import jax.numpy as jnp
from jax.experimental import pallas as pl

N = 100000
E = 1600000
G = 64


def _mlp3(h, w0, b0, w1, b1, w2, b2):
    h = jax.nn.elu(h @ w0 + b0)
    h = jax.nn.elu(h @ w1 + b1)
    return h @ w2 + b2


def _final_kernel(xmax_ref, w_ref, b_ref, o_ref):
    o_ref[...] = xmax_ref[...] @ w_ref[...] + b_ref[...]


def kernel(x, pos, edge_index, batch, l1w0, l1b0, l1w1, l1b1, l1w2, l1b2,
           g1w0, g1b0, g1w1, g1b1, g1w2, g1b2, l2w0, l2b0, l2w1, l2b1,
           l2w2, l2b2, g2w0, g2b0, g2w1, g2b1, g2w2, g2b2, linw, linb):
    loops = jnp.arange(N, dtype=edge_index.dtype)
    src = jnp.concatenate([edge_index[0], loops])
    dst = jnp.concatenate([edge_index[1], loops])
    h = jnp.concatenate([x[src], pos[src] - pos[dst]], axis=1)
    h = _mlp3(h, l1w0, l1b0, l1w1, l1b1, l1w2, l1b2)
    a1 = jax.ops.segment_max(h, dst, num_segments=N)
    x1 = jax.nn.elu(_mlp3(a1, g1w0, g1b0, g1w1, g1b1, g1w2, g1b2))
    h2 = jnp.concatenate([x1[src], pos[src] - pos[dst]], axis=1)
    h2 = _mlp3(h2, l2w0, l2b0, l2w1, l2b1, l2w2, l2b2)
    a2 = jax.ops.segment_max(h2, dst, num_segments=N)
    x2 = jax.nn.elu(_mlp3(a2, g2w0, g2b0, g2w1, g2b1, g2w2, g2b2))
    x_max = jax.ops.segment_max(x2, batch, num_segments=G)
    out = pl.pallas_call(
        _final_kernel,
        out_shape=jax.ShapeDtypeStruct((G, 2), jnp.float32),
    )(x_max, linw, linb[None, :])
    return out


# speedup vs baseline: 1.0854x; 1.0824x over previous
import functools

import jax
import jax.numpy as jnp
from jax import lax
from jax.experimental import pallas as pl
from jax.experimental.pallas import tpu as pltpu
from jax.experimental.pallas import tpu_sc as plsc

N = 100000
E = 1600000
G = 64

NW = 32
EP = E + N
CHUNK = 2048
EP_PAD = 1703936
NP = 102400
BE = 8192
BN = 2048
BP = 1024
NEG = -3.0e38


def _sc_gather(table, idx, d):
    b_per_w = EP_PAD // NW
    n_chunks = b_per_w // CHUNK
    mesh = plsc.VectorSubcoreMesh(core_axis_name="c", subcore_axis_name="s")

    @functools.partial(
        pl.kernel,
        mesh=mesh,
        compiler_params=pltpu.CompilerParams(use_tc_tiling_on_sc=False),
        out_type=jax.ShapeDtypeStruct((EP_PAD, d), jnp.float32),
        scratch_types=[
            pltpu.VMEM((CHUNK,), jnp.int32),
            pltpu.VMEM((CHUNK, d), jnp.float32),
            pltpu.SemaphoreType.DMA,
        ],
    )
    def k(table_hbm, idx_hbm, out_hbm, idx_v, rows_v, sem):
        wid = lax.axis_index("s") * 2 + lax.axis_index("c")
        base = wid * b_per_w

        def body(i, carry):
            off = base + i * CHUNK
            pltpu.sync_copy(idx_hbm.at[pl.ds(off, CHUNK)], idx_v)
            pltpu.async_copy(table_hbm.at[idx_v], rows_v, sem).wait()
            pltpu.sync_copy(rows_v, out_hbm.at[pl.ds(off, CHUNK)])
            return carry

        lax.fori_loop(0, n_chunks, body, 0)

    return k(table, idx)


def _dot(a, b):
    return jnp.dot(a, b, precision=lax.Precision.HIGHEST,
                   preferred_element_type=jnp.float32)


def _elu(v):
    return jnp.where(v > 0, v, jnp.exp(jnp.minimum(v, 0.0)) - 1.0)


def _edge_mlp_kernel(s_ref, d_ref, ws_ref, wd_ref, b0_ref, w1_ref, b1_ref,
                     w2_ref, b2_ref, o_ref):
    h = _dot(s_ref[...], ws_ref[...]) + _dot(d_ref[...], wd_ref[...]) + b0_ref[...]
    h = _elu(h)
    h = _elu(_dot(h, w1_ref[...]) + b1_ref[...])
    o_ref[...] = _dot(h, w2_ref[...]) + b2_ref[...]


def _edge_mlp(s, dmat, ws, wd, b0, w1, b1, w2, b2, dw):
    grid = EP_PAD // BE
    full = lambda i: (0, 0)
    return pl.pallas_call(
        _edge_mlp_kernel,
        grid=(grid,),
        in_specs=[
            pl.BlockSpec((BE, s.shape[1]), lambda i: (i, 0)),
            pl.BlockSpec((BE, 8), lambda i: (i, 0)),
            pl.BlockSpec(ws.shape, full),
            pl.BlockSpec(wd.shape, full),
            pl.BlockSpec(b0.shape, full),
            pl.BlockSpec(w1.shape, full),
            pl.BlockSpec(b1.shape, full),
            pl.BlockSpec(w2.shape, full),
            pl.BlockSpec(b2.shape, full),
        ],
        out_specs=pl.BlockSpec((BE, dw), lambda i: (i, 0)),
        out_shape=jax.ShapeDtypeStruct((EP_PAD, dw), jnp.float32),
    )(s, dmat, ws, wd, b0, w1, b1, w2, b2)


def _g1_kernel(a_ref, p_ref, w0_ref, b0_ref, w1_ref, b1_ref, w2_ref, b2_ref,
               o_ref):
    h = _elu(_dot(a_ref[...], w0_ref[...]) + b0_ref[...])
    h = _elu(_dot(h, w1_ref[...]) + b1_ref[...])
    x1 = _elu(_dot(h, w2_ref[...]) + b2_ref[...])
    o_ref[...] = jnp.concatenate(
        [x1, p_ref[...], jnp.zeros((BN, 4), jnp.float32)], axis=1)


def _g1(a1, posp, w0, b0, w1, b1, w2, b2):
    grid = NP // BN
    full = lambda i: (0, 0)
    return pl.pallas_call(
        _g1_kernel,
        grid=(grid,),
        in_specs=[
            pl.BlockSpec((BN, 32), lambda i: (i, 0)),
            pl.BlockSpec((BN, 4), lambda i: (i, 0)),
            pl.BlockSpec(w0.shape, full),
            pl.BlockSpec(b0.shape, full),
            pl.BlockSpec(w1.shape, full),
            pl.BlockSpec(b1.shape, full),
            pl.BlockSpec(w2.shape, full),
            pl.BlockSpec(b2.shape, full),
        ],
        out_specs=pl.BlockSpec((BN, 40), lambda i: (i, 0)),
        out_shape=jax.ShapeDtypeStruct((NP, 40), jnp.float32),
    )(a1, posp, w0, b0, w1, b1, w2, b2)


def _g2_pool_kernel(a_ref, b_ref, w0_ref, b0_ref, w1_ref, b1_ref, w2_ref,
                    b2_ref, lw_ref, lb_ref, o_ref, acc_ref):
    i = pl.program_id(0)

    @pl.when(i == 0)
    def _():
        acc_ref[...] = jnp.full((G, 40), NEG, jnp.float32)

    h = _elu(_dot(a_ref[...], w0_ref[...]) + b0_ref[...])
    h = _elu(_dot(h, w1_ref[...]) + b1_ref[...])
    x2 = _elu(_dot(h, w2_ref[...]) + b2_ref[...])
    ids = b_ref[...]
    for g in range(G):
        m = ids == jnp.float32(g)
        vals = jnp.where(m, x2, NEG)
        mx = jnp.max(vals, axis=0, keepdims=True)
        acc_ref[g:g + 1, :] = jnp.maximum(acc_ref[g:g + 1, :], mx)

    @pl.when(i == pl.num_programs(0) - 1)
    def _():
        o_ref[...] = _dot(acc_ref[...], lw_ref[...]) + lb_ref[...]


def _g2_pool(a2, bidsp, w0, b0, w1, b1, w2, b2, lw, lb):
    grid = NP // BP
    full = lambda i: (0, 0)
    return pl.pallas_call(
        _g2_pool_kernel,
        grid=(grid,),
        in_specs=[
            pl.BlockSpec((BP, 40), lambda i: (i, 0)),
            pl.BlockSpec((BP, 1), lambda i: (i, 0)),
            pl.BlockSpec(w0.shape, full),
            pl.BlockSpec(b0.shape, full),
            pl.BlockSpec(w1.shape, full),
            pl.BlockSpec(b1.shape, full),
            pl.BlockSpec(w2.shape, full),
            pl.BlockSpec(b2.shape, full),
            pl.BlockSpec(lw.shape, full),
            pl.BlockSpec(lb.shape, full),
        ],
        out_specs=pl.BlockSpec((G, 2), full),
        out_shape=jax.ShapeDtypeStruct((G, 2), jnp.float32),
        scratch_shapes=[pltpu.VMEM((G, 40), jnp.float32)],
    )(a2, bidsp, w0, b0, w1, b1, w2, b2, lw, lb)


def _pad2(w, rows, cols):
    out = jnp.zeros((rows, cols), jnp.float32)
    return out.at[:w.shape[0], :w.shape[1]].set(w)


def kernel(x, pos, edge_index, batch, l1w0, l1b0, l1w1, l1b1, l1w2, l1b2,
           g1w0, g1b0, g1w1, g1b1, g1w2, g1b2, l2w0, l2b0, l2w1, l2b1,
           l2w2, l2b2, g2w0, g2b0, g2w1, g2b1, g2w2, g2b2, linw, linb):
    loops = jnp.arange(N, dtype=jnp.int32)
    padi = jnp.zeros((EP_PAD - EP,), jnp.int32)
    src = jnp.concatenate([edge_index[0].astype(jnp.int32), loops, padi])
    dst_g = jnp.concatenate([edge_index[1].astype(jnp.int32), loops, padi])
    dst_seg = jnp.concatenate(
        [edge_index[1].astype(jnp.int32), loops,
         jnp.full((EP_PAD - EP,), N, jnp.int32)])

    t1 = jnp.concatenate([x, pos, jnp.zeros((N, 2), jnp.float32)], axis=1)
    posp = jnp.concatenate(
        [pos, jnp.zeros((N, 1), jnp.float32)], axis=1)
    posp = jnp.concatenate(
        [posp, jnp.zeros((NP - N, 4), jnp.float32)], axis=0)
    bidsp = jnp.concatenate(
        [batch.astype(jnp.float32), jnp.full((NP - N,), G, jnp.float32)]
    ).reshape(NP, 1)

    ws1 = jnp.concatenate([l1w0, jnp.zeros((2, 32), jnp.float32)], axis=0)
    wd1 = jnp.concatenate(
        [jnp.zeros((3, 32), jnp.float32), -l1w0[3:6],
         jnp.zeros((2, 32), jnp.float32)], axis=0)

    sg = _sc_gather(t1, src, 8)
    dg = _sc_gather(t1, dst_g, 8)

    h1 = _edge_mlp(sg, dg, ws1, wd1, l1b0.reshape(1, 32), l1w1,
                   l1b1.reshape(1, 32), l1w2, l1b2.reshape(1, 32), 32)
    a1 = jax.ops.segment_max(h1, dst_seg, num_segments=NP)
    t2 = _g1(a1, posp, g1w0, g1b0.reshape(1, 32), g1w1, g1b1.reshape(1, 32),
             g1w2, g1b2.reshape(1, 32))

    ws2 = _pad2(l2w0, 40, 40)
    wd2 = jnp.zeros((8, 40), jnp.float32).at[3:6, :35].set(-l2w0[32:35])
    w1p = _pad2(l2w1, 40, 40)
    w2p = _pad2(l2w2, 40, 40)
    b0p = _pad2(l2b0.reshape(1, 35), 1, 40)
    b1p = _pad2(l2b1.reshape(1, 35), 1, 40)
    b2p = _pad2(l2b2.reshape(1, 35), 1, 40)

    s2 = _sc_gather(t2, src, 40)
    h2 = _edge_mlp(s2, dg, ws2, wd2, b0p, w1p, b1p, w2p, b2p, 40)
    a2 = jax.ops.segment_max(h2, dst_seg, num_segments=NP)

    gw0 = _pad2(g2w0, 40, 40)
    gw1 = _pad2(g2w1, 40, 40)
    gw2 = _pad2(g2w2, 40, 40)
    gb0 = _pad2(g2b0.reshape(1, 35), 1, 40)
    gb1 = _pad2(g2b1.reshape(1, 35), 1, 40)
    gb2 = _pad2(g2b2.reshape(1, 35), 1, 40)
    lwp = _pad2(linw, 40, 2)

    out = _g2_pool(a2, bidsp, gw0, gb0, gw1, gb1, gw2, gb2, lwp,
                   linb.reshape(1, 2))
    return out


# speedup vs baseline: 1.0860x; 1.0005x over previous
import functools

import jax
import jax.numpy as jnp
from jax import lax
from jax.experimental import pallas as pl
from jax.experimental.pallas import tpu as pltpu
from jax.experimental.pallas import tpu_sc as plsc

N = 100000
E = 1600000
G = 64

NW = 32
EP = E + N
CHUNK = 2048
EP_PAD = 1703936
NP = 102400
BE = 8192
BN = 2048
BP = 1024
NEG = -3.0e38


def _sc_gather(table, idx, d):
    chunk = 4096 if d <= 16 else 2048
    b_per_w = EP_PAD // NW
    n_chunks = b_per_w // chunk
    mesh = plsc.VectorSubcoreMesh(core_axis_name="c", subcore_axis_name="s")

    @functools.partial(
        pl.kernel,
        mesh=mesh,
        compiler_params=pltpu.CompilerParams(use_tc_tiling_on_sc=False),
        out_type=jax.ShapeDtypeStruct((EP_PAD, d), jnp.float32),
        scratch_types=[
            pltpu.VMEM((chunk,), jnp.int32),
            pltpu.VMEM((chunk, d), jnp.float32),
            pltpu.SemaphoreType.DMA,
        ],
    )
    def k(table_hbm, idx_hbm, out_hbm, idx_v, rows_v, sem):
        wid = lax.axis_index("s") * 2 + lax.axis_index("c")
        base = wid * b_per_w

        def body(i, carry):
            off = base + i * chunk
            pltpu.sync_copy(idx_hbm.at[pl.ds(off, chunk)], idx_v)
            pltpu.async_copy(table_hbm.at[idx_v], rows_v, sem).wait()
            pltpu.sync_copy(rows_v, out_hbm.at[pl.ds(off, chunk)])
            return carry

        lax.fori_loop(0, n_chunks, body, 0)

    return k(table, idx)


def _dot(a, b):
    return jnp.dot(a, b, precision=lax.Precision.HIGHEST,
                   preferred_element_type=jnp.float32)


def _elu(v):
    return jnp.where(v > 0, v, jnp.exp(jnp.minimum(v, 0.0)) - 1.0)


def _edge_mlp_kernel(s_ref, d_ref, ws_ref, wd_ref, b0_ref, w1_ref, b1_ref,
                     w2_ref, b2_ref, o_ref):
    h = _dot(s_ref[...], ws_ref[...]) + _dot(d_ref[...], wd_ref[...]) + b0_ref[...]
    h = _elu(h)
    h = _elu(_dot(h, w1_ref[...]) + b1_ref[...])
    o_ref[...] = _dot(h, w2_ref[...]) + b2_ref[...]


def _edge_mlp(s, dmat, ws, wd, b0, w1, b1, w2, b2, dw):
    grid = EP_PAD // BE
    full = lambda i: (0, 0)
    return pl.pallas_call(
        _edge_mlp_kernel,
        grid=(grid,),
        in_specs=[
            pl.BlockSpec((BE, s.shape[1]), lambda i: (i, 0)),
            pl.BlockSpec((BE, 8), lambda i: (i, 0)),
            pl.BlockSpec(ws.shape, full),
            pl.BlockSpec(wd.shape, full),
            pl.BlockSpec(b0.shape, full),
            pl.BlockSpec(w1.shape, full),
            pl.BlockSpec(b1.shape, full),
            pl.BlockSpec(w2.shape, full),
            pl.BlockSpec(b2.shape, full),
        ],
        out_specs=pl.BlockSpec((BE, dw), lambda i: (i, 0)),
        out_shape=jax.ShapeDtypeStruct((EP_PAD, dw), jnp.float32),
    )(s, dmat, ws, wd, b0, w1, b1, w2, b2)


def _g1_kernel(a_ref, p_ref, w0_ref, b0_ref, w1_ref, b1_ref, w2_ref, b2_ref,
               o_ref):
    h = _elu(_dot(a_ref[...], w0_ref[...]) + b0_ref[...])
    h = _elu(_dot(h, w1_ref[...]) + b1_ref[...])
    x1 = _elu(_dot(h, w2_ref[...]) + b2_ref[...])
    o_ref[...] = jnp.concatenate(
        [x1, p_ref[...], jnp.zeros((BN, 4), jnp.float32)], axis=1)


def _g1(a1, posp, w0, b0, w1, b1, w2, b2):
    grid = NP // BN
    full = lambda i: (0, 0)
    return pl.pallas_call(
        _g1_kernel,
        grid=(grid,),
        in_specs=[
            pl.BlockSpec((BN, 32), lambda i: (i, 0)),
            pl.BlockSpec((BN, 4), lambda i: (i, 0)),
            pl.BlockSpec(w0.shape, full),
            pl.BlockSpec(b0.shape, full),
            pl.BlockSpec(w1.shape, full),
            pl.BlockSpec(b1.shape, full),
            pl.BlockSpec(w2.shape, full),
            pl.BlockSpec(b2.shape, full),
        ],
        out_specs=pl.BlockSpec((BN, 40), lambda i: (i, 0)),
        out_shape=jax.ShapeDtypeStruct((NP, 40), jnp.float32),
    )(a1, posp, w0, b0, w1, b1, w2, b2)


def _g2_pool_kernel(a_ref, b_ref, w0_ref, b0_ref, w1_ref, b1_ref, w2_ref,
                    b2_ref, lw_ref, lb_ref, o_ref, acc_ref):
    i = pl.program_id(0)

    @pl.when(i == 0)
    def _():
        acc_ref[...] = jnp.full((G, 40), NEG, jnp.float32)

    h = _elu(_dot(a_ref[...], w0_ref[...]) + b0_ref[...])
    h = _elu(_dot(h, w1_ref[...]) + b1_ref[...])
    x2 = _elu(_dot(h, w2_ref[...]) + b2_ref[...])
    ids = b_ref[...]
    for g in range(G):
        m = ids == jnp.float32(g)
        vals = jnp.where(m, x2, NEG)
        mx = jnp.max(vals, axis=0, keepdims=True)
        acc_ref[g:g + 1, :] = jnp.maximum(acc_ref[g:g + 1, :], mx)

    @pl.when(i == pl.num_programs(0) - 1)
    def _():
        o_ref[...] = _dot(acc_ref[...], lw_ref[...]) + lb_ref[...]


def _g2_pool(a2, bidsp, w0, b0, w1, b1, w2, b2, lw, lb):
    grid = NP // BP
    full = lambda i: (0, 0)
    return pl.pallas_call(
        _g2_pool_kernel,
        grid=(grid,),
        in_specs=[
            pl.BlockSpec((BP, 40), lambda i: (i, 0)),
            pl.BlockSpec((BP, 1), lambda i: (i, 0)),
            pl.BlockSpec(w0.shape, full),
            pl.BlockSpec(b0.shape, full),
            pl.BlockSpec(w1.shape, full),
            pl.BlockSpec(b1.shape, full),
            pl.BlockSpec(w2.shape, full),
            pl.BlockSpec(b2.shape, full),
            pl.BlockSpec(lw.shape, full),
            pl.BlockSpec(lb.shape, full),
        ],
        out_specs=pl.BlockSpec((G, 2), full),
        out_shape=jax.ShapeDtypeStruct((G, 2), jnp.float32),
        scratch_shapes=[pltpu.VMEM((G, 40), jnp.float32)],
    )(a2, bidsp, w0, b0, w1, b1, w2, b2, lw, lb)


def _pad2(w, rows, cols):
    out = jnp.zeros((rows, cols), jnp.float32)
    return out.at[:w.shape[0], :w.shape[1]].set(w)


def kernel(x, pos, edge_index, batch, l1w0, l1b0, l1w1, l1b1, l1w2, l1b2,
           g1w0, g1b0, g1w1, g1b1, g1w2, g1b2, l2w0, l2b0, l2w1, l2b1,
           l2w2, l2b2, g2w0, g2b0, g2w1, g2b1, g2w2, g2b2, linw, linb):
    loops = jnp.arange(N, dtype=jnp.int32)
    padi = jnp.zeros((EP_PAD - EP,), jnp.int32)
    src = jnp.concatenate([edge_index[0].astype(jnp.int32), loops, padi])
    dst_g = jnp.concatenate([edge_index[1].astype(jnp.int32), loops, padi])
    dst_seg = jnp.concatenate(
        [edge_index[1].astype(jnp.int32), loops,
         jnp.full((EP_PAD - EP,), N, jnp.int32)])

    t1 = jnp.concatenate([x, pos, jnp.zeros((N, 2), jnp.float32)], axis=1)
    posp = jnp.concatenate(
        [pos, jnp.zeros((N, 1), jnp.float32)], axis=1)
    posp = jnp.concatenate(
        [posp, jnp.zeros((NP - N, 4), jnp.float32)], axis=0)
    bidsp = jnp.concatenate(
        [batch.astype(jnp.float32), jnp.full((NP - N,), G, jnp.float32)]
    ).reshape(NP, 1)

    ws1 = jnp.concatenate([l1w0, jnp.zeros((2, 32), jnp.float32)], axis=0)
    wd1 = jnp.concatenate(
        [jnp.zeros((3, 32), jnp.float32), -l1w0[3:6],
         jnp.zeros((2, 32), jnp.float32)], axis=0)

    sg = _sc_gather(t1, src, 8)
    dg = _sc_gather(t1, dst_g, 8)

    h1 = _edge_mlp(sg, dg, ws1, wd1, l1b0.reshape(1, 32), l1w1,
                   l1b1.reshape(1, 32), l1w2, l1b2.reshape(1, 32), 32)
    a1 = jax.ops.segment_max(h1, dst_seg, num_segments=NP)
    t2 = _g1(a1, posp, g1w0, g1b0.reshape(1, 32), g1w1, g1b1.reshape(1, 32),
             g1w2, g1b2.reshape(1, 32))

    ws2 = _pad2(l2w0, 40, 40)
    wd2 = jnp.zeros((8, 40), jnp.float32).at[3:6, :35].set(-l2w0[32:35])
    w1p = _pad2(l2w1, 40, 40)
    w2p = _pad2(l2w2, 40, 40)
    b0p = _pad2(l2b0.reshape(1, 35), 1, 40)
    b1p = _pad2(l2b1.reshape(1, 35), 1, 40)
    b2p = _pad2(l2b2.reshape(1, 35), 1, 40)

    s2 = _sc_gather(t2, src, 40)
    h2 = _edge_mlp(s2, dg, ws2, wd2, b0p, w1p, b1p, w2p, b2p, 40)
    a2 = jax.ops.segment_max(h2, dst_seg, num_segments=NP)

    gw0 = _pad2(g2w0, 40, 40)
    gw1 = _pad2(g2w1, 40, 40)
    gw2 = _pad2(g2w2, 40, 40)
    gb0 = _pad2(g2b0.reshape(1, 35), 1, 40)
    gb1 = _pad2(g2b1.reshape(1, 35), 1, 40)
    gb2 = _pad2(g2b2.reshape(1, 35), 1, 40)
    lwp = _pad2(linw, 40, 2)

    out = _g2_pool(a2, bidsp, gw0, gb0, gw1, gb1, gw2, gb2, lwp,
                   linb.reshape(1, 2))
    return out
